# Initial kernel scaffold; baseline (speedup 1.0000x reference)
import jax
import jax.numpy as jnp
from jax.experimental import pallas as pl


def kernel(x, W_ih_f, W_hh_f, b_ih_f, b_hh_f, W_ih_b, W_hh_b, b_ih_b, b_hh_b, gate_W, gate_b, W1, b1, W2, b2):
    raise NotImplementedError("write your pallas kernel here")


# speedup vs baseline: 5.4864x; 5.4864x over previous
import functools

import jax
import jax.numpy as jnp
from jax.experimental import pallas as pl
from jax.experimental.pallas import tpu as pltpu

D_MODEL = 768
H = 1536
G3 = 3 * H
GRU_OUT = 2 * H
T = 2048
NUM_EXPERTS = 8
FFN_DIM = 2048
CAPACITY = 512

_HI = jax.lax.Precision.HIGHEST
_BF = jnp.bfloat16


def _bdot(a, b, dims):
    return jax.lax.dot_general(a.astype(_BF), b.astype(_BF), dims,
                               preferred_element_type=jnp.float32)


def _xproj_body(x_ref, wt_hbm, b_ref, o_ref, w_s, sem):
    @pl.when(pl.program_id(0) == 0)
    def _():
        pltpu.make_async_copy(wt_hbm, w_s, sem).start()
        pltpu.make_async_copy(wt_hbm, w_s, sem).wait()

    acc = _bdot(x_ref[...], w_s[...], (((1,), (0,)), ((), ())))
    o_ref[...] = acc + b_ref[...]


def _xproj(x2d, wt_cat, b_cat):
    m_blk = 256
    grid = (T // m_blk,)
    return pl.pallas_call(
        _xproj_body,
        grid=grid,
        in_specs=[
            pl.BlockSpec((m_blk, D_MODEL), lambda m: (m, 0)),
            pl.BlockSpec(memory_space=pltpu.MemorySpace.HBM),
            pl.BlockSpec((1, 2 * G3), lambda m: (0, 0)),
        ],
        out_specs=pl.BlockSpec((m_blk, 2 * G3), lambda m: (m, 0)),
        out_shape=jax.ShapeDtypeStruct((T, 2 * G3), jnp.float32),
        scratch_shapes=[pltpu.VMEM((D_MODEL, 2 * G3), jnp.float32),
                        pltpu.SemaphoreType.DMA],
    )(x2d, wt_cat, b_cat)


def _gru_body(xp_ref, wt_hbm, bhh_ref, o_ref, h_s, w_s, sem, *, steps, reverse):
    c = pl.program_id(0)

    @pl.when(c == 0)
    def _():
        pltpu.make_async_copy(wt_hbm, w_s, sem).start()
        pltpu.make_async_copy(wt_hbm, w_s, sem).wait()
        h_s[...] = jnp.zeros_like(h_s)

    bhh = bhh_ref[...]

    def step(i, h_row):
        local = (steps - 1 - i) if reverse else i
        gh = jax.lax.dot_general(h_row.astype(_BF), w_s[...],
                                 (((1,), (0,)), ((), ())),
                                 preferred_element_type=jnp.float32) + bhh
        xp_i = xp_ref[pl.ds(local, 1), :]
        r = jax.nn.sigmoid(xp_i[:, :H] + gh[:, :H])
        z = jax.nn.sigmoid(xp_i[:, H:2 * H] + gh[:, H:2 * H])
        n = jnp.tanh(xp_i[:, 2 * H:] + r * gh[:, 2 * H:])
        h_new = (1.0 - z) * n + z * h_row
        o_ref[pl.ds(local, 1), :] = jnp.where(h_new >= 0, h_new, 0.01 * h_new)
        return h_new

    h = jax.lax.fori_loop(0, steps, step, h_s[...])
    h_s[...] = h


def _gru_scan(xp, wt_hh, b_hh, col, reverse):
    s_blk = 128
    grid = (T // s_blk,)
    nc = T // s_blk
    if reverse:
        xmap = lambda c: (nc - 1 - c, col)
        omap = lambda c: (nc - 1 - c, 0)
    else:
        xmap = lambda c: (c, col)
        omap = lambda c: (c, 0)
    body = functools.partial(_gru_body, steps=s_blk, reverse=reverse)
    return pl.pallas_call(
        body,
        grid=grid,
        in_specs=[
            pl.BlockSpec((s_blk, G3), xmap),
            pl.BlockSpec(memory_space=pltpu.MemorySpace.HBM),
            pl.BlockSpec((1, G3), lambda c: (0, 0)),
        ],
        out_specs=pl.BlockSpec((s_blk, H), omap),
        out_shape=jax.ShapeDtypeStruct((T, H), jnp.float32),
        scratch_shapes=[pltpu.VMEM((1, H), jnp.float32),
                        pltpu.VMEM((H, G3), _BF),
                        pltpu.SemaphoreType.DMA],
    )(xp, wt_hh, b_hh)


def _route_body(flat_ref, gw_ref, gb_ref, w_ref):
    flat = flat_ref[...]
    logits = _bdot(flat, gw_ref[...],
                   (((1,), (1,)), ((), ()))) + gb_ref[...]
    m = jnp.max(logits, axis=1, keepdims=True)
    ex = jnp.exp(logits - m)
    scores = ex / jnp.sum(ex, axis=1, keepdims=True)
    scores_t = scores.T

    i_blk = 256
    j_iota = jax.lax.broadcasted_iota(jnp.int32, (1, T), 1)
    for e in range(NUM_EXPERTS):
        row = scores_t[e:e + 1, :]
        for ci in range(T // i_blk):
            col = scores[ci * i_blk:(ci + 1) * i_blk, e:e + 1]
            i_iota = jax.lax.broadcasted_iota(
                jnp.int32, (i_blk, 1), 0) + ci * i_blk
            beats = (row > col) | ((row == col) & (j_iota < i_iota))
            rank = jnp.sum(beats.astype(jnp.int32), axis=1, keepdims=True)
            w_ref[pl.ds(ci * i_blk, i_blk), e:e + 1] = jnp.where(
                rank < CAPACITY, col, 0.0)


def _route(flat, gate_w, gate_b):
    return pl.pallas_call(
        _route_body,
        in_specs=[
            pl.BlockSpec((T, GRU_OUT), lambda: (0, 0)),
            pl.BlockSpec((NUM_EXPERTS, GRU_OUT), lambda: (0, 0)),
            pl.BlockSpec((1, NUM_EXPERTS), lambda: (0, 0)),
        ],
        out_specs=pl.BlockSpec((T, NUM_EXPERTS), lambda: (0, 0)),
        out_shape=jax.ShapeDtypeStruct((T, NUM_EXPERTS), jnp.float32),
    )(flat, gate_w, gate_b)


def _ffn_body(flat_ref, w1_ref, b1_ref, w2_ref, b2_ref, w_ref, y_ref, *, nf):
    e = pl.program_id(1)
    f = pl.program_id(2)

    @pl.when((e == 0) & (f == 0))
    def _():
        y_ref[...] = jnp.zeros_like(y_ref)

    onehot = (jax.lax.broadcasted_iota(jnp.int32, (1, NUM_EXPERTS), 1) == e)
    wcol = jnp.sum(w_ref[...] * onehot.astype(jnp.float32), axis=1,
                   keepdims=True)

    xb = flat_ref[...]
    h = _bdot(xb, w1_ref[0], (((1,), (1,)), ((), ()))) + b1_ref[0]
    h = 0.5 * h * (1.0 + jax.lax.erf(h * (2.0 ** -0.5)))
    part = _bdot(h, w2_ref[0], (((1,), (1,)), ((), ())))
    contrib = part

    @pl.when(f == 0)
    def _():
        y_ref[...] += wcol * (contrib + b2_ref[0])

    @pl.when(f != 0)
    def _():
        y_ref[...] += wcol * contrib


def _ffn(flat, w1, b1, w2, b2, w2d):
    f_blk = 512
    m_blk = 512
    nf = FFN_DIM // f_blk
    body = functools.partial(_ffn_body, nf=nf)
    return pl.pallas_call(
        body,
        grid=(T // m_blk, NUM_EXPERTS, nf),
        in_specs=[
            pl.BlockSpec((m_blk, GRU_OUT), lambda m, e, f: (m, 0)),
            pl.BlockSpec((1, f_blk, GRU_OUT), lambda m, e, f: (e, f, 0)),
            pl.BlockSpec((1, 1, f_blk), lambda m, e, f: (e, 0, f)),
            pl.BlockSpec((1, D_MODEL, f_blk), lambda m, e, f: (e, 0, f)),
            pl.BlockSpec((1, 1, D_MODEL), lambda m, e, f: (e, 0, 0)),
            pl.BlockSpec((m_blk, NUM_EXPERTS), lambda m, e, f: (m, 0)),
        ],
        out_specs=pl.BlockSpec((m_blk, D_MODEL), lambda m, e, f: (m, 0)),
        out_shape=jax.ShapeDtypeStruct((T, D_MODEL), jnp.float32),
    )(flat, w1, b1, w2, b2, w2d)


def kernel(x, W_ih_f, W_hh_f, b_ih_f, b_hh_f, W_ih_b, W_hh_b, b_ih_b, b_hh_b,
           gate_W, gate_b, W1, b1, W2, b2):
    xs = jnp.swapaxes(x, 0, 1)
    xp_f = (xs @ W_ih_f.T + b_ih_f).reshape(T, G3)
    xp_b = (xs @ W_ih_b.T + b_ih_b).reshape(T, G3)
    xp = jnp.concatenate([xp_f, xp_b], axis=1)

    out_f = _gru_scan(xp, W_hh_f.T.astype(_BF), b_hh_f.reshape(1, G3), 0, False)
    out_b = _gru_scan(xp, W_hh_b.T.astype(_BF), b_hh_b.reshape(1, G3), 1, True)
    flat = jnp.concatenate([out_f, out_b], axis=1)

    w2d = _route(flat, gate_W, gate_b.reshape(1, NUM_EXPERTS))

    y2d = _ffn(flat, W1, b1.reshape(NUM_EXPERTS, 1, FFN_DIM),
               W2, b2.reshape(NUM_EXPERTS, 1, D_MODEL), w2d)
    return (y2d.reshape(1, T, D_MODEL), jnp.float32(0.0))
